# baseline (speedup 1.0000x reference)
import jax
import jax.numpy as jnp
from jax import lax
from jax.experimental import pallas as pl
from jax.experimental.pallas import tpu as pltpu
from jax.experimental.pallas import tpu_sc as plsc

B, N, D = 1024, 1000, 128
NQ = 16
CTX = 2 * D

_info = plsc.get_sparse_core_info()
NC, NS = _info.num_cores, _info.num_subcores
NW = NC * NS
BPW = B // NW
CHUNK = 8
NCHUNKS = BPW // CHUNK
CROWS = CHUNK * NQ
NBUF = 3


def _tec_body(emb_hbm, aux_hbm, w_hbm, out_hbm,
              aux_v, w_v, ifa_v, icn_v, o_v,
              gs0, gs1, gs2, ws0, ws1, ws2):
    wid = lax.axis_index("s") * NC + lax.axis_index("c")
    b0 = wid * BPW
    gsems = [gs0, gs1, gs2]
    wsems = [ws0, ws1, ws2]

    pltpu.sync_copy(aux_hbm.at[pl.ds(b0, BPW)], aux_v)
    pltpu.sync_copy(w_hbm, w_v)
    phv = [w_v[v] for v in range(16)]

    def build_idx(c):
        for jj in range(CHUNK):
            j = c * CHUNK + jj
            base = (b0 + j) * N
            ifa_v[c, pl.ds(jj * 16, 16)] = aux_v[j, pl.ds(0, 16)] + base
            icn_v[c, pl.ds(jj * 16, 16)] = aux_v[j, pl.ds(16, 16)] + base

    def fire_gather(c):
        buf = c % NBUF
        return [
            pltpu.async_copy(emb_hbm.at[ifa_v.at[c]],
                             o_v.at[buf, pl.ds(0, CROWS), pl.ds(0, 128)],
                             gsems[buf]),
            pltpu.async_copy(emb_hbm.at[icn_v.at[c]],
                             o_v.at[buf, pl.ds(0, CROWS), pl.ds(128, 128)],
                             gsems[buf]),
        ]

    gh = {}
    for c in range(2):
        build_idx(c)
        gh[c] = fire_gather(c)
    for c in range(2, NCHUNKS):
        build_idx(c)
    wh = {}
    for c in range(NCHUNKS):
        buf = c % NBUF
        if c + 2 < NCHUNKS:
            if c - 1 in wh:
                for h in wh.pop(c - 1):
                    h.wait()
            gh[c + 2] = fire_gather(c + 2)
        for h in gh.pop(c):
            h.wait()

        @plsc.parallel_loop(0, CROWS, unroll=4)
        def _blend(t):
            jb = c * CHUNK + t // NQ
            swb = aux_v[jb, pl.ds(32, 16)] != 0
            for v in range(16):
                x = o_v[buf, t, pl.ds(v * 16, 16)]
                o_v[buf, t, pl.ds(v * 16, 16)] = jnp.where(swb, phv[v], x)

        wh[c] = [pltpu.async_copy(
            o_v.at[buf],
            out_hbm.at[pl.ds((b0 + c * CHUNK) * NQ, CROWS)],
            wsems[buf])]
    for c in list(wh):
        for h in wh.pop(c):
            h.wait()


@jax.jit
def _tsp_context_sc(emb_flat, aux, w16):
    mesh = plsc.VectorSubcoreMesh(core_axis_name="c", subcore_axis_name="s")
    run = pl.kernel(
        _tec_body,
        mesh=mesh,
        out_type=jax.ShapeDtypeStruct((B * NQ, CTX), jnp.float32),
        scratch_types=[
            pltpu.VMEM((BPW, 48), jnp.int32),
            pltpu.VMEM((16, 16), jnp.float32),
            pltpu.VMEM((NCHUNKS, CROWS), jnp.int32),
            pltpu.VMEM((NCHUNKS, CROWS), jnp.int32),
            pltpu.VMEM((NBUF, CROWS, CTX), jnp.float32),
            pltpu.SemaphoreType.DMA,
            pltpu.SemaphoreType.DMA,
            pltpu.SemaphoreType.DMA,
            pltpu.SemaphoreType.DMA,
            pltpu.SemaphoreType.DMA,
            pltpu.SemaphoreType.DMA,
        ],
    )
    return run(emb_flat, aux, w16)


def kernel(embeddings, first_a, current_node, is_initial_action, W_placeholder):
    emb_flat = embeddings.reshape(B * N, D)
    aux = jnp.concatenate([
        first_a.astype(jnp.int32),
        current_node.astype(jnp.int32),
        jnp.broadcast_to(is_initial_action.astype(jnp.int32)[:, None], (B, 16)),
    ], axis=1)
    w16 = W_placeholder.reshape(16, 16)
    out = _tsp_context_sc(emb_flat, aux, w16)
    return out.reshape(B, NQ, CTX)


# speedup vs baseline: 1.0120x; 1.0120x over previous
import jax
import jax.numpy as jnp
from jax import lax
from jax.experimental import pallas as pl
from jax.experimental.pallas import tpu as pltpu
from jax.experimental.pallas import tpu_sc as plsc

B, N, D = 1024, 1000, 128
NQ = 16
CTX = 2 * D

_info = plsc.get_sparse_core_info()
NC, NS = _info.num_cores, _info.num_subcores
NW = NC * NS
BPW = B // NW
CHUNK = 8
NCHUNKS = BPW // CHUNK
CROWS = CHUNK * NQ
NBUF = 3


def _tec_body(emb_hbm, aux_hbm, w_hbm, out_hbm,
              aux_v, w_v, ifa_v, icn_v, o_v,
              gs0, gs1, gs2, ws0, ws1, ws2):
    wid = lax.axis_index("s") * NC + lax.axis_index("c")
    b0 = wid * BPW
    gsems = [gs0, gs1, gs2]
    wsems = [ws0, ws1, ws2]

    pltpu.sync_copy(aux_hbm.at[pl.ds(b0, BPW)], aux_v)
    pltpu.sync_copy(w_hbm, w_v)
    phv = [w_v[v] for v in range(16)]

    def build_idx(c):
        for jj in range(CHUNK):
            j = c * CHUNK + jj
            base = (b0 + j) * N
            ifa_v[c, pl.ds(jj * 16, 16)] = aux_v[j, pl.ds(0, 16)] + base
            icn_v[c, pl.ds(jj * 16, 16)] = aux_v[j, pl.ds(16, 16)] + base

    def fire_gather(c):
        buf = c % NBUF
        return [
            pltpu.async_copy(emb_hbm.at[ifa_v.at[c]],
                             o_v.at[buf, pl.ds(0, CROWS), pl.ds(0, 128)],
                             gsems[buf]),
            pltpu.async_copy(emb_hbm.at[icn_v.at[c]],
                             o_v.at[buf, pl.ds(0, CROWS), pl.ds(128, 128)],
                             gsems[buf]),
        ]

    gh = {}
    for c in range(2):
        build_idx(c)
        gh[c] = fire_gather(c)
    for c in range(2, NCHUNKS):
        build_idx(c)
    wh = {}
    for c in range(NCHUNKS):
        buf = c % NBUF
        if c + 2 < NCHUNKS:
            if c - 1 in wh:
                for h in wh.pop(c - 1):
                    h.wait()
            gh[c + 2] = fire_gather(c + 2)
        for h in gh.pop(c):
            h.wait()

        @plsc.parallel_loop(0, CROWS, unroll=2)
        def _blend(t):
            jb = c * CHUNK + t // NQ
            swb = aux_v[jb, pl.ds(32, 16)] != 0
            for v in range(16):
                x = o_v[buf, t, pl.ds(v * 16, 16)]
                o_v[buf, t, pl.ds(v * 16, 16)] = jnp.where(swb, phv[v], x)

        wh[c] = [pltpu.async_copy(
            o_v.at[buf],
            out_hbm.at[pl.ds((b0 + c * CHUNK) * NQ, CROWS)],
            wsems[buf])]
    for c in list(wh):
        for h in wh.pop(c):
            h.wait()


@jax.jit
def _tsp_context_sc(emb_flat, aux, w16):
    mesh = plsc.VectorSubcoreMesh(core_axis_name="c", subcore_axis_name="s")
    run = pl.kernel(
        _tec_body,
        mesh=mesh,
        out_type=jax.ShapeDtypeStruct((B * NQ, CTX), jnp.float32),
        scratch_types=[
            pltpu.VMEM((BPW, 48), jnp.int32),
            pltpu.VMEM((16, 16), jnp.float32),
            pltpu.VMEM((NCHUNKS, CROWS), jnp.int32),
            pltpu.VMEM((NCHUNKS, CROWS), jnp.int32),
            pltpu.VMEM((NBUF, CROWS, CTX), jnp.float32),
            pltpu.SemaphoreType.DMA,
            pltpu.SemaphoreType.DMA,
            pltpu.SemaphoreType.DMA,
            pltpu.SemaphoreType.DMA,
            pltpu.SemaphoreType.DMA,
            pltpu.SemaphoreType.DMA,
        ],
    )
    return run(emb_flat, aux, w16)


def kernel(embeddings, first_a, current_node, is_initial_action, W_placeholder):
    emb_flat = embeddings.reshape(B * N, D)
    aux = jnp.concatenate([
        first_a.astype(jnp.int32),
        current_node.astype(jnp.int32),
        jnp.broadcast_to(is_initial_action.astype(jnp.int32)[:, None], (B, 16)),
    ], axis=1)
    w16 = W_placeholder.reshape(16, 16)
    out = _tsp_context_sc(emb_flat, aux, w16)
    return out.reshape(B, NQ, CTX)


# speedup vs baseline: 1.0193x; 1.0072x over previous
import jax
import jax.numpy as jnp
from jax import lax
from jax.experimental import pallas as pl
from jax.experimental.pallas import tpu as pltpu
from jax.experimental.pallas import tpu_sc as plsc

B, N, D = 1024, 1000, 128
NQ = 16
CTX = 2 * D

_info = plsc.get_sparse_core_info()
NC, NS = _info.num_cores, _info.num_subcores
NW = NC * NS
BPW = B // NW
CHUNK = 8
NCHUNKS = BPW // CHUNK
CROWS = CHUNK * NQ
NBUF = 3


def _tec_body(emb_hbm, fa2_hbm, cn_hbm, w_hbm, out_hbm,
              fa2_v, cn_v, w_v, ifa_v, icn_v, o_v,
              gs0, gs1, gs2, ws0, ws1, ws2):
    wid = lax.axis_index("s") * NC + lax.axis_index("c")
    b0 = wid * BPW
    gsems = [gs0, gs1, gs2]
    wsems = [ws0, ws1, ws2]

    prolog = [
        pltpu.async_copy(fa2_hbm.at[pl.ds(b0, BPW)], fa2_v, gs0),
        pltpu.async_copy(cn_hbm.at[pl.ds(b0, BPW)], cn_v, gs1),
        pltpu.async_copy(w_hbm, w_v, gs2),
    ]
    for h in prolog:
        h.wait()
    phv = [w_v[pl.ds(v * 16, 16)] for v in range(16)]

    def build_idx(c):
        for jj in range(CHUNK):
            j = c * CHUNK + jj
            base = (b0 + j) * N
            ifa_v[c, pl.ds(jj * 16, 16)] = (fa2_v[j] & 0x3FFFFFFF) + base
            icn_v[c, pl.ds(jj * 16, 16)] = cn_v[j] + base

    def fire_gather(c):
        buf = c % NBUF
        return [
            pltpu.async_copy(emb_hbm.at[ifa_v.at[c]],
                             o_v.at[buf, pl.ds(0, CROWS), pl.ds(0, 128)],
                             gsems[buf]),
            pltpu.async_copy(emb_hbm.at[icn_v.at[c]],
                             o_v.at[buf, pl.ds(0, CROWS), pl.ds(128, 128)],
                             gsems[buf]),
        ]

    gh = {}
    for c in range(2):
        build_idx(c)
        gh[c] = fire_gather(c)
    for c in range(2, NCHUNKS):
        build_idx(c)
    wh = {}
    for c in range(NCHUNKS):
        buf = c % NBUF
        if c + 2 < NCHUNKS:
            if c - 1 in wh:
                for h in wh.pop(c - 1):
                    h.wait()
            gh[c + 2] = fire_gather(c + 2)
        for h in gh.pop(c):
            h.wait()

        @plsc.parallel_loop(0, CROWS, unroll=2)
        def _blend(t):
            jb = c * CHUNK + t // NQ
            swb = fa2_v[jb] >= (1 << 30)
            for v in range(16):
                x = o_v[buf, t, pl.ds(v * 16, 16)]
                o_v[buf, t, pl.ds(v * 16, 16)] = jnp.where(swb, phv[v], x)

        wh[c] = [pltpu.async_copy(
            o_v.at[buf],
            out_hbm.at[pl.ds((b0 + c * CHUNK) * NQ, CROWS)],
            wsems[buf])]
    for c in list(wh):
        for h in wh.pop(c):
            h.wait()


@jax.jit
def _tsp_context_sc(emb_flat, fa2, cn, w):
    mesh = plsc.VectorSubcoreMesh(core_axis_name="c", subcore_axis_name="s")
    run = pl.kernel(
        _tec_body,
        mesh=mesh,
        out_type=jax.ShapeDtypeStruct((B * NQ, CTX), jnp.float32),
        scratch_types=[
            pltpu.VMEM((BPW, NQ), jnp.int32),
            pltpu.VMEM((BPW, NQ), jnp.int32),
            pltpu.VMEM((256,), jnp.float32),
            pltpu.VMEM((NCHUNKS, CROWS), jnp.int32),
            pltpu.VMEM((NCHUNKS, CROWS), jnp.int32),
            pltpu.VMEM((NBUF, CROWS, CTX), jnp.float32),
            pltpu.SemaphoreType.DMA,
            pltpu.SemaphoreType.DMA,
            pltpu.SemaphoreType.DMA,
            pltpu.SemaphoreType.DMA,
            pltpu.SemaphoreType.DMA,
            pltpu.SemaphoreType.DMA,
        ],
    )
    return run(emb_flat, fa2, cn, w)


def kernel(embeddings, first_a, current_node, is_initial_action, W_placeholder):
    emb_flat = embeddings.reshape(B * N, D)
    fa2 = first_a.astype(jnp.int32) | (
        is_initial_action.astype(jnp.int32)[:, None] << 30)
    out = _tsp_context_sc(emb_flat, fa2, current_node.astype(jnp.int32),
                          W_placeholder)
    return out.reshape(B, NQ, CTX)


# speedup vs baseline: 1.0508x; 1.0309x over previous
import jax
import jax.numpy as jnp
from jax import lax
from jax.experimental import pallas as pl
from jax.experimental.pallas import tpu as pltpu
from jax.experimental.pallas import tpu_sc as plsc

B, N, D = 1024, 1000, 128
NQ = 16
CTX = 2 * D

_info = plsc.get_sparse_core_info()
NC, NS = _info.num_cores, _info.num_subcores
NW = NC * NS
BPW = B // NW
CHUNK = 4
NCHUNKS = BPW // CHUNK
CROWS = CHUNK * NQ
NBUF = 6
LOOKAHEAD = 5


def _tec_body(emb_hbm, fa2_hbm, cn_hbm, w_hbm, out_hbm,
              fa2_v, cn_v, w_v, ifa_v, icn_v, o_v,
              gs0, gs1, gs2, gs3, gs4, gs5,
              ws0, ws1, ws2, ws3, ws4, ws5):
    wid = lax.axis_index("s") * NC + lax.axis_index("c")
    b0 = wid * BPW
    gsems = [gs0, gs1, gs2, gs3, gs4, gs5]
    wsems = [ws0, ws1, ws2, ws3, ws4, ws5]

    prolog = [
        pltpu.async_copy(fa2_hbm.at[pl.ds(b0, BPW)], fa2_v, gs0),
        pltpu.async_copy(cn_hbm.at[pl.ds(b0, BPW)], cn_v, gs1),
        pltpu.async_copy(w_hbm, w_v, gs2),
    ]
    for h in prolog:
        h.wait()
    phv = [w_v[pl.ds(v * 16, 16)] for v in range(16)]

    def build_idx(c):
        for jj in range(CHUNK):
            j = c * CHUNK + jj
            base = (b0 + j) * N
            ifa_v[c, pl.ds(jj * 16, 16)] = (fa2_v[j] & 0x3FFFFFFF) + base
            icn_v[c, pl.ds(jj * 16, 16)] = cn_v[j] + base

    def fire_gather(c):
        buf = c % NBUF
        return [
            pltpu.async_copy(emb_hbm.at[ifa_v.at[c]],
                             o_v.at[buf, pl.ds(0, CROWS), pl.ds(0, 128)],
                             gsems[buf]),
            pltpu.async_copy(emb_hbm.at[icn_v.at[c]],
                             o_v.at[buf, pl.ds(0, CROWS), pl.ds(128, 128)],
                             gsems[buf]),
        ]

    gh = {}
    for c in range(LOOKAHEAD):
        build_idx(c)
        gh[c] = fire_gather(c)
    for c in range(LOOKAHEAD, NCHUNKS):
        build_idx(c)
    wh = {}
    for c in range(NCHUNKS):
        buf = c % NBUF
        if c + LOOKAHEAD < NCHUNKS:
            if c - 1 in wh:
                for h in wh.pop(c - 1):
                    h.wait()
            gh[c + LOOKAHEAD] = fire_gather(c + LOOKAHEAD)
        for h in gh.pop(c):
            h.wait()

        @plsc.parallel_loop(0, CROWS, unroll=2)
        def _blend(t):
            jb = c * CHUNK + t // NQ
            swb = fa2_v[jb] >= (1 << 30)
            for v in range(16):
                x = o_v[buf, t, pl.ds(v * 16, 16)]
                o_v[buf, t, pl.ds(v * 16, 16)] = jnp.where(swb, phv[v], x)

        wh[c] = [pltpu.async_copy(
            o_v.at[buf],
            out_hbm.at[pl.ds((b0 + c * CHUNK) * NQ, CROWS)],
            wsems[buf])]
    for c in list(wh):
        for h in wh.pop(c):
            h.wait()


@jax.jit
def _tsp_context_sc(emb_flat, fa2, cn, w):
    mesh = plsc.VectorSubcoreMesh(core_axis_name="c", subcore_axis_name="s")
    run = pl.kernel(
        _tec_body,
        mesh=mesh,
        out_type=jax.ShapeDtypeStruct((B * NQ, CTX), jnp.float32),
        scratch_types=[
            pltpu.VMEM((BPW, NQ), jnp.int32),
            pltpu.VMEM((BPW, NQ), jnp.int32),
            pltpu.VMEM((256,), jnp.float32),
            pltpu.VMEM((NCHUNKS, CROWS), jnp.int32),
            pltpu.VMEM((NCHUNKS, CROWS), jnp.int32),
            pltpu.VMEM((NBUF, CROWS, CTX), jnp.float32),
        ] + [pltpu.SemaphoreType.DMA] * 12 + [
        ],
    )
    return run(emb_flat, fa2, cn, w)


def kernel(embeddings, first_a, current_node, is_initial_action, W_placeholder):
    emb_flat = embeddings.reshape(B * N, D)
    fa2 = first_a.astype(jnp.int32) | (
        is_initial_action.astype(jnp.int32)[:, None] << 30)
    out = _tsp_context_sc(emb_flat, fa2, current_node.astype(jnp.int32),
                          W_placeholder)
    return out.reshape(B, NQ, CTX)
